# baseline (speedup 1.0000x reference)
import functools

import jax
import jax.numpy as jnp
import numpy as np
from jax import lax
from jax.experimental import pallas as pl
from jax.experimental.pallas import tpu as pltpu
from jax.experimental.pallas import tpu_sc as plsc

NN = 10000
EE = 320000
HH = 128
CHUNK = 128
NCHUNKS = EE // CHUNK
NCORES = 2
NSUB = 16
NTILES = NCORES * NSUB
ROWS_PER_TILE = 624
TAIL_ROW0 = ROWS_PER_TILE * NSUB
TAIL_ROWS = NN - TAIL_ROW0
SPAN = 80
HSPAN = SPAN // 2
NCHP = SPAN * NTILES
DSPAN = NCHP // NSUB
NACC = NN + 32
NDUP = 15
RB = 1000
NBLK = NN // RB
EPSV = 1e-5

_F32 = jnp.float32


def _vmesh():
    return plsc.VectorSubcoreMesh(core_axis_name="c", subcore_axis_name="s")


def _sc_degrees(src2d, dst2d, zvec):
    @functools.partial(
        pl.kernel,
        out_type=[jax.ShapeDtypeStruct((NN,), _F32),
                  jax.ShapeDtypeStruct((NN,), _F32)],
        mesh=_vmesh(),
        scratch_types=[
            pltpu.VMEM_SHARED((NN,), _F32),
            pltpu.VMEM((DSPAN, CHUNK), jnp.int32),
            pltpu.VMEM((CHUNK,), _F32),
        ],
    )
    def k(src_hbm, dst_hbm, z_hbm, od_hbm, id_hbm, acc_sh, idx_v, ones_v):
        c = lax.axis_index("c")
        s = lax.axis_index("s")
        j0 = s * DSPAN

        @pl.loop(0, CHUNK // 16)
        def _(j):
            ones_v[pl.ds(j * 16, 16)] = jnp.full((16,), 1.0, _F32)

        @pl.when(c == 0)
        def _():
            pltpu.sync_copy(src_hbm.at[pl.ds(j0, DSPAN)], idx_v)

        @pl.when(c == 1)
        def _():
            pltpu.sync_copy(dst_hbm.at[pl.ds(j0, DSPAN)], idx_v)

        @pl.when(s == 0)
        def _():
            pltpu.sync_copy(z_hbm, acc_sh)

        plsc.subcore_barrier()

        @pl.loop(0, DSPAN)
        def _(i):
            @pl.when(j0 + i < NCHUNKS)
            def _():
                pltpu.sync_copy(ones_v, acc_sh.at[idx_v.at[i]], add=True)

        plsc.subcore_barrier()

        @pl.when(s == 0)
        def _():
            @pl.when(c == 0)
            def _():
                pltpu.sync_copy(acc_sh, od_hbm)

            @pl.when(c == 1)
            def _():
                pltpu.sync_copy(acc_sh, id_hbm)

    return k(src2d, dst2d, zvec)


def _sc_aggregate(m, src2d, dmask2d, dupinfo, zrows):
    @functools.partial(
        pl.kernel,
        out_type=jax.ShapeDtypeStruct((NCORES, NN, HH), _F32),
        mesh=_vmesh(),
        scratch_types=[
            pltpu.VMEM_SHARED((NACC, HH), _F32),
            pltpu.VMEM((HSPAN, CHUNK), jnp.int32),
            pltpu.VMEM((HSPAN, CHUNK), jnp.int32),
            pltpu.VMEM((CHUNK, HH), _F32),
            pltpu.VMEM((CHUNK, HH), _F32),
            pltpu.VMEM((HSPAN * (NDUP + 1),), jnp.int32),
            pltpu.SemaphoreType.DMA,
            pltpu.SemaphoreType.DMA,
        ],
    )
    def k(m_hbm, src_hbm, dst_hbm, info_hbm, z_hbm, out_hbm, acc_sh, si_v,
          di_v, rows0, rows1, info_v, gsem0, gsem1):
        c = lax.axis_index("c")
        s = lax.axis_index("s")
        tid = c * NSUB + s
        j0 = tid * SPAN

        r0 = s * ROWS_PER_TILE
        pltpu.sync_copy(
            z_hbm.at[pl.ds(r0, ROWS_PER_TILE)],
            acc_sh.at[pl.ds(r0, ROWS_PER_TILE)],
        )

        @pl.when(s == NSUB - 1)
        def _():
            pltpu.sync_copy(
                z_hbm.at[pl.ds(TAIL_ROW0, NACC - TAIL_ROW0)],
                acc_sh.at[pl.ds(TAIL_ROW0, NACC - TAIL_ROW0)],
            )

        plsc.subcore_barrier()

        def _roll_dups(rows, i):
            iv = info_v[pl.ds(i * (NDUP + 1), 16)]
            cnt = iv[0]
            for kk in range(NDUP):
                @pl.when(kk < cnt)
                def _(kk=kk):
                    v = iv[kk + 1]
                    fr = jax.lax.shift_right_logical(v, 8)
                    to = jax.lax.bitwise_and(v, 255)
                    for q in range(HH // 16):
                        rows[to, pl.ds(q * 16, 16)] = (
                            rows[to, pl.ds(q * 16, 16)]
                            + rows[fr, pl.ds(q * 16, 16)]
                        )

        def _half(base):
            pltpu.sync_copy(src_hbm.at[pl.ds(base, HSPAN)], si_v)
            pltpu.sync_copy(dst_hbm.at[pl.ds(base, HSPAN)], di_v)
            pltpu.sync_copy(
                info_hbm.at[pl.ds(base * (NDUP + 1), HSPAN * (NDUP + 1))],
                info_v)

            @pl.when(base < NCHUNKS)
            def _():
                pltpu.async_copy(m_hbm.at[si_v.at[0]], rows0, gsem0)

            @pl.when(base + 1 < NCHUNKS)
            def _():
                pltpu.async_copy(m_hbm.at[si_v.at[1]], rows1, gsem1)

            @pl.loop(0, HSPAN, step=2)
            def _(i):
                @pl.when(base + i < NCHUNKS)
                def _():
                    pltpu.make_async_copy(
                        m_hbm.at[si_v.at[i]], rows0, gsem0).wait()
                    _roll_dups(rows0, i)
                    pltpu.sync_copy(rows0, acc_sh.at[di_v.at[i]], add=True)

                    @pl.when(jnp.logical_and(i + 2 < HSPAN,
                                             base + i + 2 < NCHUNKS))
                    def _():
                        pltpu.async_copy(m_hbm.at[si_v.at[i + 2]], rows0, gsem0)

                @pl.when(base + i + 1 < NCHUNKS)
                def _():
                    pltpu.make_async_copy(
                        m_hbm.at[si_v.at[i + 1]], rows1, gsem1).wait()
                    _roll_dups(rows1, i + 1)
                    pltpu.sync_copy(rows1, acc_sh.at[di_v.at[i + 1]], add=True)

                    @pl.when(jnp.logical_and(i + 3 < HSPAN,
                                             base + i + 3 < NCHUNKS))
                    def _():
                        pltpu.async_copy(m_hbm.at[si_v.at[i + 3]], rows1, gsem1)

        _half(j0)
        _half(j0 + HSPAN)

        plsc.subcore_barrier()

        pltpu.sync_copy(
            acc_sh.at[pl.ds(r0, ROWS_PER_TILE)],
            out_hbm.at[c].at[pl.ds(r0, ROWS_PER_TILE)],
        )

        @pl.when(s == NSUB - 1)
        def _():
            pltpu.sync_copy(
                acc_sh.at[pl.ds(TAIL_ROW0, TAIL_ROWS)],
                out_hbm.at[c].at[pl.ds(TAIL_ROW0, TAIL_ROWS)],
            )

    return k(m, src2d, dmask2d, dupinfo, zrows)


def _tc_dupinfo(dst2d):
    def body(d_ref, mask_ref, info_ref):
        d = d_ref[...]
        lane = jax.lax.broadcasted_iota(jnp.int32, d.shape, 1)
        lastocc = lane
        for sh in range(1, CHUNK):
            shifted = jnp.pad(d[:, sh:], ((0, 0), (0, sh)),
                              constant_values=-1)
            lastocc = jnp.where(d == shifted, lane + sh, lastocc)
        isdup = lastocc > lane
        mask_ref[...] = jnp.where(isdup, NN + (lane % 32), d)
        c = isdup.astype(jnp.int32)
        for sh in (1, 2, 4, 8, 16, 32, 64):
            c = c + jnp.pad(c[:, :-sh], ((0, 0), (sh, 0)))
        slot = c - 1
        packed = lane * 256 + lastocc
        cols = [jnp.minimum(c[:, -1:], NDUP)]
        for k in range(NDUP):
            sel = jnp.where(jnp.logical_and(isdup, slot == k), packed, 0)
            cols.append(jnp.sum(sel, axis=1, keepdims=True).astype(jnp.int32))
        info_ref[...] = jnp.concatenate(cols, axis=1)

    DB = 128
    return pl.pallas_call(
        body,
        grid=(NCHP // DB,),
        in_specs=[pl.BlockSpec((DB, CHUNK), lambda i: (i, 0))],
        out_specs=[pl.BlockSpec((DB, CHUNK), lambda i: (i, 0)),
                   pl.BlockSpec((DB, NDUP + 1), lambda i: (i, 0))],
        out_shape=[jax.ShapeDtypeStruct((NCHP, CHUNK), jnp.int32),
                   jax.ShapeDtypeStruct((NCHP, NDUP + 1), jnp.int32)],
    )(dst2d)


def _tc_xw(x, W):
    def body(x_ref, w_ref, o_ref):
        o_ref[...] = jnp.dot(
            x_ref[...], w_ref[...], preferred_element_type=_F32,
            precision=jax.lax.Precision.HIGHEST
        )

    return pl.pallas_call(
        body,
        grid=(NBLK,),
        in_specs=[
            pl.BlockSpec((RB, HH), lambda i: (i, 0)),
            pl.BlockSpec((HH, HH), lambda i: (0, 0)),
        ],
        out_specs=pl.BlockSpec((RB, HH), lambda i: (i, 0)),
        out_shape=jax.ShapeDtypeStruct((NN, HH), _F32),
    )(x, W)


def _tc_norms_scale(od, idg, xw):
    def body(od_ref, id_ref, xw_ref, ns_ref, nd_ref, m_ref):
        ns = lax.rsqrt(jnp.maximum(od_ref[...], 1.0))
        ns_ref[...] = ns
        nd_ref[...] = lax.rsqrt(jnp.maximum(id_ref[...], 1.0))
        m_ref[...] = xw_ref[...] * ns

    return pl.pallas_call(
        body,
        grid=(NBLK,),
        in_specs=[
            pl.BlockSpec((RB, 1), lambda i: (i, 0)),
            pl.BlockSpec((RB, 1), lambda i: (i, 0)),
            pl.BlockSpec((RB, HH), lambda i: (i, 0)),
        ],
        out_specs=[
            pl.BlockSpec((RB, 1), lambda i: (i, 0)),
            pl.BlockSpec((RB, 1), lambda i: (i, 0)),
            pl.BlockSpec((RB, HH), lambda i: (i, 0)),
        ],
        out_shape=[jax.ShapeDtypeStruct((NN, 1), _F32),
                   jax.ShapeDtypeStruct((NN, 1), _F32),
                   jax.ShapeDtypeStruct((NN, HH), _F32)],
    )(od, idg, xw)


def _tc_mid(p, nd, b, ns, W):
    def body(p_ref, nd_ref, b_ref, ns_ref, w_ref, o_ref):
        pr = p_ref[...]
        h = jnp.maximum((pr[0] + pr[1]) * nd_ref[...] + b_ref[...], 0.0)
        o_ref[...] = jnp.dot(
            h * ns_ref[...], w_ref[...], preferred_element_type=_F32,
            precision=jax.lax.Precision.HIGHEST
        )

    return pl.pallas_call(
        body,
        grid=(NBLK,),
        in_specs=[
            pl.BlockSpec((NCORES, RB, HH), lambda i: (0, i, 0)),
            pl.BlockSpec((RB, 1), lambda i: (i, 0)),
            pl.BlockSpec((1, HH), lambda i: (0, 0)),
            pl.BlockSpec((RB, 1), lambda i: (i, 0)),
            pl.BlockSpec((HH, HH), lambda i: (0, 0)),
        ],
        out_specs=pl.BlockSpec((RB, HH), lambda i: (i, 0)),
        out_shape=jax.ShapeDtypeStruct((NN, HH), _F32),
    )(p, nd, b, ns, W)


def _tc_final(p, nd, b, w1, b1, g, be, w2, b2):
    def body(p_ref, nd_ref, b_ref, w1_ref, b1_ref, g_ref, be_ref, w2_ref,
             b2_ref, o_ref, sacc, macc):
        i = pl.program_id(0)
        pr = p_ref[...]
        h = jnp.maximum((pr[0] + pr[1]) * nd_ref[...] + b_ref[...], 0.0)
        bs = jnp.sum(h, axis=0, keepdims=True)
        bm = jnp.max(h, axis=0, keepdims=True)

        @pl.when(i == 0)
        def _():
            sacc[...] = bs
            macc[...] = bm

        @pl.when(i > 0)
        def _():
            sacc[...] += bs
            macc[...] = jnp.maximum(macc[...], bm)

        @pl.when(i == NBLK - 1)
        def _():
            rs = sacc[...]
            rm = rs * (1.0 / NN)
            rx = macc[...]
            w1r = w1_ref[...]
            z = (
                jnp.dot(rs, w1r[0:HH], preferred_element_type=_F32, precision=jax.lax.Precision.HIGHEST)
                + jnp.dot(rm, w1r[HH:2 * HH], preferred_element_type=_F32, precision=jax.lax.Precision.HIGHEST)
                + jnp.dot(rx, w1r[2 * HH:3 * HH], preferred_element_type=_F32, precision=jax.lax.Precision.HIGHEST)
                + b1_ref[...]
            )
            z = z * (g_ref[...] * float(1.0 / np.sqrt(1.0 + EPSV))) + be_ref[...]
            z = jnp.maximum(z, 0.0)
            o_ref[...] = (
                jnp.dot(z, w2_ref[...], preferred_element_type=_F32, precision=jax.lax.Precision.HIGHEST)
                + b2_ref[...]
            )

    return pl.pallas_call(
        body,
        grid=(NBLK,),
        in_specs=[
            pl.BlockSpec((NCORES, RB, HH), lambda i: (0, i, 0)),
            pl.BlockSpec((RB, 1), lambda i: (i, 0)),
            pl.BlockSpec((1, HH), lambda i: (0, 0)),
            pl.BlockSpec((3 * HH, HH), lambda i: (0, 0)),
            pl.BlockSpec((1, HH), lambda i: (0, 0)),
            pl.BlockSpec((1, HH), lambda i: (0, 0)),
            pl.BlockSpec((1, HH), lambda i: (0, 0)),
            pl.BlockSpec((HH, 1), lambda i: (0, 0)),
            pl.BlockSpec((1, 1), lambda i: (0, 0)),
        ],
        out_specs=pl.BlockSpec((1, 1), lambda i: (0, 0)),
        out_shape=jax.ShapeDtypeStruct((1, 1), _F32),
        scratch_shapes=[
            pltpu.VMEM((1, HH), _F32),
            pltpu.VMEM((1, HH), _F32),
        ],
    )(p, nd, b, w1, b1, g, be, w2, b2)


def kernel(x, edge_index, W0, b0, W1, b1, W2, b2, W3, b3, W4, b4,
           mlpW1, mlpb1, gamma, beta, mlpW2, mlpb2):
    zvec = jnp.zeros((NN,), _F32)
    zrows = jnp.zeros((NACC, HH), _F32)
    pad = jnp.zeros((NCHP * CHUNK - EE,), jnp.int32)
    src2d = jnp.concatenate([edge_index[0], pad]).reshape(NCHP, CHUNK)
    dst2d = jnp.concatenate([edge_index[1], pad]).reshape(NCHP, CHUNK)

    dmask2d, dupinfo = _tc_dupinfo(dst2d)
    dupflat = dupinfo.reshape(NCHP * (NDUP + 1))
    xw = _tc_xw(x, W0)
    od, idg = _sc_degrees(src2d, dst2d, zvec)
    ns, nd, m = _tc_norms_scale(od.reshape(NN, 1), idg.reshape(NN, 1), xw)

    Ws = [W0, W1, W2, W3, W4]
    bs = [b0.reshape(1, HH), b1.reshape(1, HH), b2.reshape(1, HH),
          b3.reshape(1, HH), b4.reshape(1, HH)]

    p = None
    for l in range(5):
        p = _sc_aggregate(m, src2d, dmask2d, dupflat, zrows)
        if l < 4:
            m = _tc_mid(p, nd, bs[l], ns, Ws[l + 1])

    return _tc_final(
        p, nd, bs[4], mlpW1, mlpb1.reshape(1, HH), gamma.reshape(1, HH),
        beta.reshape(1, HH), mlpW2, mlpb2.reshape(1, 1),
    )


# speedup vs baseline: 1.0133x; 1.0133x over previous
import functools

import jax
import jax.numpy as jnp
import numpy as np
from jax import lax
from jax.experimental import pallas as pl
from jax.experimental.pallas import tpu as pltpu
from jax.experimental.pallas import tpu_sc as plsc

NN = 10000
EE = 320000
HH = 128
CHUNK = 128
NCHUNKS = EE // CHUNK
NCORES = 2
NSUB = 16
NTILES = NCORES * NSUB
ROWS_PER_TILE = 624
TAIL_ROW0 = ROWS_PER_TILE * NSUB
TAIL_ROWS = NN - TAIL_ROW0
SPAN = 80
HSPAN = SPAN // 2
NCHP = SPAN * NTILES
DSPAN = NCHP // NSUB
NACC = NN + 32
NDUP = 15
RB = 1000
NBLK = NN // RB
EPSV = 1e-5

_F32 = jnp.float32


def _vmesh():
    return plsc.VectorSubcoreMesh(core_axis_name="c", subcore_axis_name="s")


def _sc_degrees(src2d, dst2d, zvec):
    @functools.partial(
        pl.kernel,
        out_type=[jax.ShapeDtypeStruct((NN,), _F32),
                  jax.ShapeDtypeStruct((NN,), _F32)],
        mesh=_vmesh(),
        scratch_types=[
            pltpu.VMEM_SHARED((NN,), _F32),
            pltpu.VMEM((DSPAN, CHUNK), jnp.int32),
            pltpu.VMEM((CHUNK,), _F32),
        ],
    )
    def k(src_hbm, dst_hbm, z_hbm, od_hbm, id_hbm, acc_sh, idx_v, ones_v):
        c = lax.axis_index("c")
        s = lax.axis_index("s")
        j0 = s * DSPAN

        @pl.loop(0, CHUNK // 16)
        def _(j):
            ones_v[pl.ds(j * 16, 16)] = jnp.full((16,), 1.0, _F32)

        @pl.when(c == 0)
        def _():
            pltpu.sync_copy(src_hbm.at[pl.ds(j0, DSPAN)], idx_v)

        @pl.when(c == 1)
        def _():
            pltpu.sync_copy(dst_hbm.at[pl.ds(j0, DSPAN)], idx_v)

        @pl.when(s == 0)
        def _():
            pltpu.sync_copy(z_hbm, acc_sh)

        plsc.subcore_barrier()

        @pl.loop(0, DSPAN)
        def _(i):
            @pl.when(j0 + i < NCHUNKS)
            def _():
                pltpu.sync_copy(ones_v, acc_sh.at[idx_v.at[i]], add=True)

        plsc.subcore_barrier()

        @pl.when(s == 0)
        def _():
            @pl.when(c == 0)
            def _():
                pltpu.sync_copy(acc_sh, od_hbm)

            @pl.when(c == 1)
            def _():
                pltpu.sync_copy(acc_sh, id_hbm)

    return k(src2d, dst2d, zvec)


def _sc_aggregate(m, src2d, dmask2d, dupinfo, zrows):
    @functools.partial(
        pl.kernel,
        out_type=jax.ShapeDtypeStruct((NCORES, NN, HH), _F32),
        mesh=_vmesh(),
        scratch_types=[
            pltpu.VMEM_SHARED((NACC, HH), _F32),
            pltpu.VMEM((HSPAN, CHUNK), jnp.int32),
            pltpu.VMEM((HSPAN, CHUNK), jnp.int32),
            pltpu.VMEM((CHUNK, HH), _F32),
            pltpu.VMEM((CHUNK, HH), _F32),
            pltpu.VMEM((HSPAN * (NDUP + 1),), jnp.int32),
            pltpu.SemaphoreType.DMA,
            pltpu.SemaphoreType.DMA,
        ],
    )
    def k(m_hbm, src_hbm, dst_hbm, info_hbm, z_hbm, out_hbm, acc_sh, si_v,
          di_v, rows0, rows1, info_v, gsem0, gsem1):
        c = lax.axis_index("c")
        s = lax.axis_index("s")
        tid = c * NSUB + s
        j0 = tid * SPAN

        r0 = s * ROWS_PER_TILE
        pltpu.sync_copy(
            z_hbm.at[pl.ds(r0, ROWS_PER_TILE)],
            acc_sh.at[pl.ds(r0, ROWS_PER_TILE)],
        )

        @pl.when(s == NSUB - 1)
        def _():
            pltpu.sync_copy(
                z_hbm.at[pl.ds(TAIL_ROW0, NACC - TAIL_ROW0)],
                acc_sh.at[pl.ds(TAIL_ROW0, NACC - TAIL_ROW0)],
            )

        plsc.subcore_barrier()

        def _roll_dups(rows, i):
            iv = info_v[pl.ds(i * (NDUP + 1), 16)]
            cnt = iv[0]
            for kk in range(NDUP):
                @pl.when(kk < cnt)
                def _(kk=kk):
                    v = iv[kk + 1]
                    fr = jax.lax.shift_right_logical(v, 8)
                    to = jax.lax.bitwise_and(v, 255)
                    for q in range(HH // 16):
                        rows[to, pl.ds(q * 16, 16)] = (
                            rows[to, pl.ds(q * 16, 16)]
                            + rows[fr, pl.ds(q * 16, 16)]
                        )

        def _half(base):
            pltpu.sync_copy(src_hbm.at[pl.ds(base, HSPAN)], si_v)
            pltpu.sync_copy(dst_hbm.at[pl.ds(base, HSPAN)], di_v)
            pltpu.sync_copy(
                info_hbm.at[pl.ds(base * (NDUP + 1), HSPAN * (NDUP + 1))],
                info_v)

            @pl.when(base < NCHUNKS)
            def _():
                pltpu.async_copy(m_hbm.at[si_v.at[0]], rows0, gsem0)

            @pl.when(base + 1 < NCHUNKS)
            def _():
                pltpu.async_copy(m_hbm.at[si_v.at[1]], rows1, gsem1)

            @pl.loop(0, HSPAN, step=2)
            def _(i):
                @pl.when(base + i < NCHUNKS)
                def _():
                    pltpu.make_async_copy(
                        m_hbm.at[si_v.at[i]], rows0, gsem0).wait()
                    _roll_dups(rows0, i)
                    pltpu.sync_copy(rows0, acc_sh.at[di_v.at[i]], add=True)

                    @pl.when(jnp.logical_and(i + 2 < HSPAN,
                                             base + i + 2 < NCHUNKS))
                    def _():
                        pltpu.async_copy(m_hbm.at[si_v.at[i + 2]], rows0, gsem0)

                @pl.when(base + i + 1 < NCHUNKS)
                def _():
                    pltpu.make_async_copy(
                        m_hbm.at[si_v.at[i + 1]], rows1, gsem1).wait()
                    _roll_dups(rows1, i + 1)
                    pltpu.sync_copy(rows1, acc_sh.at[di_v.at[i + 1]], add=True)

                    @pl.when(jnp.logical_and(i + 3 < HSPAN,
                                             base + i + 3 < NCHUNKS))
                    def _():
                        pltpu.async_copy(m_hbm.at[si_v.at[i + 3]], rows1, gsem1)

        _half(j0)
        _half(j0 + HSPAN)

        plsc.subcore_barrier()

        pltpu.sync_copy(
            acc_sh.at[pl.ds(r0, ROWS_PER_TILE)],
            out_hbm.at[c].at[pl.ds(r0, ROWS_PER_TILE)],
        )

        @pl.when(s == NSUB - 1)
        def _():
            pltpu.sync_copy(
                acc_sh.at[pl.ds(TAIL_ROW0, TAIL_ROWS)],
                out_hbm.at[c].at[pl.ds(TAIL_ROW0, TAIL_ROWS)],
            )

    return k(m, src2d, dmask2d, dupinfo, zrows)


def _tc_dupinfo(dst2d):
    def body(d_ref, mask_ref, info_ref):
        d = d_ref[...]
        lane = jax.lax.broadcasted_iota(jnp.int32, d.shape, 1)
        lastocc = lane
        for sh in range(1, CHUNK):
            shifted = jnp.pad(d[:, sh:], ((0, 0), (0, sh)),
                              constant_values=-1)
            lastocc = jnp.where(d == shifted, lane + sh, lastocc)
        isdup = lastocc > lane
        mask_ref[...] = jnp.where(isdup, NN + (lane % 32), d)
        c = isdup.astype(jnp.int32)
        for sh in (1, 2, 4, 8, 16, 32, 64):
            c = c + jnp.pad(c[:, :-sh], ((0, 0), (sh, 0)))
        slot = c - 1
        packed = lane * 256 + lastocc
        cols = [jnp.minimum(c[:, -1:], NDUP)]
        for k in range(NDUP):
            sel = jnp.where(jnp.logical_and(isdup, slot == k), packed, 0)
            cols.append(jnp.sum(sel, axis=1, keepdims=True).astype(jnp.int32))
        info_ref[...] = jnp.concatenate(cols, axis=1)

    DB = 128
    return pl.pallas_call(
        body,
        grid=(NCHP // DB,),
        in_specs=[pl.BlockSpec((DB, CHUNK), lambda i: (i, 0))],
        out_specs=[pl.BlockSpec((DB, CHUNK), lambda i: (i, 0)),
                   pl.BlockSpec((DB, NDUP + 1), lambda i: (i, 0))],
        out_shape=[jax.ShapeDtypeStruct((NCHP, CHUNK), jnp.int32),
                   jax.ShapeDtypeStruct((NCHP, NDUP + 1), jnp.int32)],
    )(dst2d)


def _dot3(a, b):
    ah = a.astype(jnp.bfloat16)
    al = (a - ah.astype(_F32)).astype(jnp.bfloat16)
    bh = b.astype(jnp.bfloat16)
    bl = (b - bh.astype(_F32)).astype(jnp.bfloat16)
    return (jnp.dot(ah, bh, preferred_element_type=_F32)
            + jnp.dot(ah, bl, preferred_element_type=_F32)
            + jnp.dot(al, bh, preferred_element_type=_F32))


def _tc_xw(x, W):
    def body(x_ref, w_ref, o_ref):
        o_ref[...] = _dot3(x_ref[...], w_ref[...])

    return pl.pallas_call(
        body,
        grid=(NBLK,),
        in_specs=[
            pl.BlockSpec((RB, HH), lambda i: (i, 0)),
            pl.BlockSpec((HH, HH), lambda i: (0, 0)),
        ],
        out_specs=pl.BlockSpec((RB, HH), lambda i: (i, 0)),
        out_shape=jax.ShapeDtypeStruct((NN, HH), _F32),
    )(x, W)


def _tc_norms_scale(od, idg, xw):
    def body(od_ref, id_ref, xw_ref, ns_ref, nd_ref, m_ref):
        ns = lax.rsqrt(jnp.maximum(od_ref[...], 1.0))
        ns_ref[...] = ns
        nd_ref[...] = lax.rsqrt(jnp.maximum(id_ref[...], 1.0))
        m_ref[...] = xw_ref[...] * ns

    return pl.pallas_call(
        body,
        grid=(NBLK,),
        in_specs=[
            pl.BlockSpec((RB, 1), lambda i: (i, 0)),
            pl.BlockSpec((RB, 1), lambda i: (i, 0)),
            pl.BlockSpec((RB, HH), lambda i: (i, 0)),
        ],
        out_specs=[
            pl.BlockSpec((RB, 1), lambda i: (i, 0)),
            pl.BlockSpec((RB, 1), lambda i: (i, 0)),
            pl.BlockSpec((RB, HH), lambda i: (i, 0)),
        ],
        out_shape=[jax.ShapeDtypeStruct((NN, 1), _F32),
                   jax.ShapeDtypeStruct((NN, 1), _F32),
                   jax.ShapeDtypeStruct((NN, HH), _F32)],
    )(od, idg, xw)


def _tc_mid(p, nd, b, ns, W):
    def body(p_ref, nd_ref, b_ref, ns_ref, w_ref, o_ref):
        pr = p_ref[...]
        h = jnp.maximum((pr[0] + pr[1]) * nd_ref[...] + b_ref[...], 0.0)
        o_ref[...] = _dot3(h * ns_ref[...], w_ref[...])

    return pl.pallas_call(
        body,
        grid=(NBLK,),
        in_specs=[
            pl.BlockSpec((NCORES, RB, HH), lambda i: (0, i, 0)),
            pl.BlockSpec((RB, 1), lambda i: (i, 0)),
            pl.BlockSpec((1, HH), lambda i: (0, 0)),
            pl.BlockSpec((RB, 1), lambda i: (i, 0)),
            pl.BlockSpec((HH, HH), lambda i: (0, 0)),
        ],
        out_specs=pl.BlockSpec((RB, HH), lambda i: (i, 0)),
        out_shape=jax.ShapeDtypeStruct((NN, HH), _F32),
    )(p, nd, b, ns, W)


def _tc_final(p, nd, b, w1, b1, g, be, w2, b2):
    def body(p_ref, nd_ref, b_ref, w1_ref, b1_ref, g_ref, be_ref, w2_ref,
             b2_ref, o_ref, sacc, macc):
        i = pl.program_id(0)
        pr = p_ref[...]
        h = jnp.maximum((pr[0] + pr[1]) * nd_ref[...] + b_ref[...], 0.0)
        bs = jnp.sum(h, axis=0, keepdims=True)
        bm = jnp.max(h, axis=0, keepdims=True)

        @pl.when(i == 0)
        def _():
            sacc[...] = bs
            macc[...] = bm

        @pl.when(i > 0)
        def _():
            sacc[...] += bs
            macc[...] = jnp.maximum(macc[...], bm)

        @pl.when(i == NBLK - 1)
        def _():
            rs = sacc[...]
            rm = rs * (1.0 / NN)
            rx = macc[...]
            w1r = w1_ref[...]
            z = (
                jnp.dot(rs, w1r[0:HH], preferred_element_type=_F32, precision=jax.lax.Precision.HIGHEST)
                + jnp.dot(rm, w1r[HH:2 * HH], preferred_element_type=_F32, precision=jax.lax.Precision.HIGHEST)
                + jnp.dot(rx, w1r[2 * HH:3 * HH], preferred_element_type=_F32, precision=jax.lax.Precision.HIGHEST)
                + b1_ref[...]
            )
            z = z * (g_ref[...] * float(1.0 / np.sqrt(1.0 + EPSV))) + be_ref[...]
            z = jnp.maximum(z, 0.0)
            o_ref[...] = (
                jnp.dot(z, w2_ref[...], preferred_element_type=_F32, precision=jax.lax.Precision.HIGHEST)
                + b2_ref[...]
            )

    return pl.pallas_call(
        body,
        grid=(NBLK,),
        in_specs=[
            pl.BlockSpec((NCORES, RB, HH), lambda i: (0, i, 0)),
            pl.BlockSpec((RB, 1), lambda i: (i, 0)),
            pl.BlockSpec((1, HH), lambda i: (0, 0)),
            pl.BlockSpec((3 * HH, HH), lambda i: (0, 0)),
            pl.BlockSpec((1, HH), lambda i: (0, 0)),
            pl.BlockSpec((1, HH), lambda i: (0, 0)),
            pl.BlockSpec((1, HH), lambda i: (0, 0)),
            pl.BlockSpec((HH, 1), lambda i: (0, 0)),
            pl.BlockSpec((1, 1), lambda i: (0, 0)),
        ],
        out_specs=pl.BlockSpec((1, 1), lambda i: (0, 0)),
        out_shape=jax.ShapeDtypeStruct((1, 1), _F32),
        scratch_shapes=[
            pltpu.VMEM((1, HH), _F32),
            pltpu.VMEM((1, HH), _F32),
        ],
    )(p, nd, b, w1, b1, g, be, w2, b2)


def kernel(x, edge_index, W0, b0, W1, b1, W2, b2, W3, b3, W4, b4,
           mlpW1, mlpb1, gamma, beta, mlpW2, mlpb2):
    zvec = jnp.zeros((NN,), _F32)
    zrows = jnp.zeros((NACC, HH), _F32)
    pad = jnp.zeros((NCHP * CHUNK - EE,), jnp.int32)
    src2d = jnp.concatenate([edge_index[0], pad]).reshape(NCHP, CHUNK)
    dst2d = jnp.concatenate([edge_index[1], pad]).reshape(NCHP, CHUNK)

    dmask2d, dupinfo = _tc_dupinfo(dst2d)
    dupflat = dupinfo.reshape(NCHP * (NDUP + 1))
    xw = _tc_xw(x, W0)
    od, idg = _sc_degrees(src2d, dst2d, zvec)
    ns, nd, m = _tc_norms_scale(od.reshape(NN, 1), idg.reshape(NN, 1), xw)

    Ws = [W0, W1, W2, W3, W4]
    bs = [b0.reshape(1, HH), b1.reshape(1, HH), b2.reshape(1, HH),
          b3.reshape(1, HH), b4.reshape(1, HH)]

    p = None
    for l in range(5):
        p = _sc_aggregate(m, src2d, dmask2d, dupflat, zrows)
        if l < 4:
            m = _tc_mid(p, nd, bs[l], ns, Ws[l + 1])

    return _tc_final(
        p, nd, bs[4], mlpW1, mlpb1.reshape(1, HH), gamma.reshape(1, HH),
        beta.reshape(1, HH), mlpW2, mlpb2.reshape(1, 1),
    )


# speedup vs baseline: 1.2273x; 1.2112x over previous
import functools

import jax
import jax.numpy as jnp
import numpy as np
from jax import lax
from jax.experimental import pallas as pl
from jax.experimental.pallas import tpu as pltpu
from jax.experimental.pallas import tpu_sc as plsc

NN = 10000
EE = 320000
HH = 128
CHUNK = 128
NCHUNKS = EE // CHUNK
NCORES = 2
NSUB = 16
NTILES = NCORES * NSUB
ROWS_PER_TILE = 624
TAIL_ROW0 = ROWS_PER_TILE * NSUB
TAIL_ROWS = NN - TAIL_ROW0
SPAN = 80
HSPAN = SPAN // 2
NCHP = SPAN * NTILES
DSPAN = NCHP // NSUB
NACC = NN + 32
NDUP = 15
RB = 1000
NBLK = NN // RB
EPSV = 1e-5

_F32 = jnp.float32


def _vmesh():
    return plsc.VectorSubcoreMesh(core_axis_name="c", subcore_axis_name="s")


def _sc_degrees(src2d, dst2d, zvec):
    @functools.partial(
        pl.kernel,
        out_type=[jax.ShapeDtypeStruct((NN,), _F32),
                  jax.ShapeDtypeStruct((NN,), _F32)],
        mesh=_vmesh(),
        scratch_types=[
            pltpu.VMEM_SHARED((NN,), _F32),
            pltpu.VMEM((DSPAN, CHUNK), jnp.int32),
            pltpu.VMEM((CHUNK,), _F32),
        ],
    )
    def k(src_hbm, dst_hbm, z_hbm, od_hbm, id_hbm, acc_sh, idx_v, ones_v):
        c = lax.axis_index("c")
        s = lax.axis_index("s")
        j0 = s * DSPAN

        @pl.loop(0, CHUNK // 16)
        def _(j):
            ones_v[pl.ds(j * 16, 16)] = jnp.full((16,), 1.0, _F32)

        @pl.when(c == 0)
        def _():
            pltpu.sync_copy(src_hbm.at[pl.ds(j0, DSPAN)], idx_v)

        @pl.when(c == 1)
        def _():
            pltpu.sync_copy(dst_hbm.at[pl.ds(j0, DSPAN)], idx_v)

        @pl.when(s == 0)
        def _():
            pltpu.sync_copy(z_hbm, acc_sh)

        plsc.subcore_barrier()

        @pl.loop(0, DSPAN)
        def _(i):
            @pl.when(j0 + i < NCHUNKS)
            def _():
                pltpu.sync_copy(ones_v, acc_sh.at[idx_v.at[i]], add=True)

        plsc.subcore_barrier()

        @pl.when(s == 0)
        def _():
            @pl.when(c == 0)
            def _():
                pltpu.sync_copy(acc_sh, od_hbm)

            @pl.when(c == 1)
            def _():
                pltpu.sync_copy(acc_sh, id_hbm)

    return k(src2d, dst2d, zvec)


def _sc_aggregate(m, src2d, dst2d, zrows):
    @functools.partial(
        pl.kernel,
        out_type=jax.ShapeDtypeStruct((NCORES, NN, HH), _F32),
        mesh=_vmesh(),
        scratch_types=[
            pltpu.VMEM_SHARED((NACC, HH), _F32),
            pltpu.VMEM((HSPAN, CHUNK), jnp.int32),
            pltpu.VMEM((HSPAN, CHUNK), jnp.int32),
            pltpu.VMEM((CHUNK, HH), _F32),
            pltpu.VMEM((CHUNK, HH), _F32),
            pltpu.SemaphoreType.DMA,
            pltpu.SemaphoreType.DMA,
        ],
    )
    def k(m_hbm, src_hbm, dst_hbm, z_hbm, out_hbm, acc_sh, si_v,
          di_v, rows0, rows1, gsem0, gsem1):
        c = lax.axis_index("c")
        s = lax.axis_index("s")
        tid = c * NSUB + s
        j0 = tid * SPAN

        r0 = s * ROWS_PER_TILE
        pltpu.sync_copy(
            z_hbm.at[pl.ds(r0, ROWS_PER_TILE)],
            acc_sh.at[pl.ds(r0, ROWS_PER_TILE)],
        )

        @pl.when(s == NSUB - 1)
        def _():
            pltpu.sync_copy(
                z_hbm.at[pl.ds(TAIL_ROW0, NACC - TAIL_ROW0)],
                acc_sh.at[pl.ds(TAIL_ROW0, NACC - TAIL_ROW0)],
            )

        plsc.subcore_barrier()

        def _half(base):
            pltpu.sync_copy(src_hbm.at[pl.ds(base, HSPAN)], si_v)
            pltpu.sync_copy(dst_hbm.at[pl.ds(base, HSPAN)], di_v)

            @pl.when(base < NCHUNKS)
            def _():
                pltpu.async_copy(m_hbm.at[si_v.at[0]], rows0, gsem0)

            @pl.when(base + 1 < NCHUNKS)
            def _():
                pltpu.async_copy(m_hbm.at[si_v.at[1]], rows1, gsem1)

            @pl.loop(0, HSPAN, step=2)
            def _(i):
                @pl.when(base + i < NCHUNKS)
                def _():
                    pltpu.make_async_copy(
                        m_hbm.at[si_v.at[i]], rows0, gsem0).wait()
                    pltpu.sync_copy(rows0, acc_sh.at[di_v.at[i]], add=True)

                    @pl.when(jnp.logical_and(i + 2 < HSPAN,
                                             base + i + 2 < NCHUNKS))
                    def _():
                        pltpu.async_copy(m_hbm.at[si_v.at[i + 2]], rows0, gsem0)

                @pl.when(base + i + 1 < NCHUNKS)
                def _():
                    pltpu.make_async_copy(
                        m_hbm.at[si_v.at[i + 1]], rows1, gsem1).wait()
                    pltpu.sync_copy(rows1, acc_sh.at[di_v.at[i + 1]], add=True)

                    @pl.when(jnp.logical_and(i + 3 < HSPAN,
                                             base + i + 3 < NCHUNKS))
                    def _():
                        pltpu.async_copy(m_hbm.at[si_v.at[i + 3]], rows1, gsem1)

        _half(j0)
        _half(j0 + HSPAN)

        plsc.subcore_barrier()

        pltpu.sync_copy(
            acc_sh.at[pl.ds(r0, ROWS_PER_TILE)],
            out_hbm.at[c].at[pl.ds(r0, ROWS_PER_TILE)],
        )

        @pl.when(s == NSUB - 1)
        def _():
            pltpu.sync_copy(
                acc_sh.at[pl.ds(TAIL_ROW0, TAIL_ROWS)],
                out_hbm.at[c].at[pl.ds(TAIL_ROW0, TAIL_ROWS)],
            )

    return k(m, src2d, dst2d, zrows)


def _dot3(a, b):
    ah = a.astype(jnp.bfloat16)
    al = (a - ah.astype(_F32)).astype(jnp.bfloat16)
    bh = b.astype(jnp.bfloat16)
    bl = (b - bh.astype(_F32)).astype(jnp.bfloat16)
    return (jnp.dot(ah, bh, preferred_element_type=_F32)
            + jnp.dot(ah, bl, preferred_element_type=_F32)
            + jnp.dot(al, bh, preferred_element_type=_F32))


def _tc_xw(x, W):
    def body(x_ref, w_ref, o_ref):
        o_ref[...] = _dot3(x_ref[...], w_ref[...])

    return pl.pallas_call(
        body,
        grid=(NBLK,),
        in_specs=[
            pl.BlockSpec((RB, HH), lambda i: (i, 0)),
            pl.BlockSpec((HH, HH), lambda i: (0, 0)),
        ],
        out_specs=pl.BlockSpec((RB, HH), lambda i: (i, 0)),
        out_shape=jax.ShapeDtypeStruct((NN, HH), _F32),
    )(x, W)


def _tc_norms_scale(od, idg, xw):
    def body(od_ref, id_ref, xw_ref, ns_ref, nd_ref, m_ref):
        ns = lax.rsqrt(jnp.maximum(od_ref[...], 1.0))
        ns_ref[...] = ns
        nd_ref[...] = lax.rsqrt(jnp.maximum(id_ref[...], 1.0))
        m_ref[...] = xw_ref[...] * ns

    return pl.pallas_call(
        body,
        grid=(NBLK,),
        in_specs=[
            pl.BlockSpec((RB, 1), lambda i: (i, 0)),
            pl.BlockSpec((RB, 1), lambda i: (i, 0)),
            pl.BlockSpec((RB, HH), lambda i: (i, 0)),
        ],
        out_specs=[
            pl.BlockSpec((RB, 1), lambda i: (i, 0)),
            pl.BlockSpec((RB, 1), lambda i: (i, 0)),
            pl.BlockSpec((RB, HH), lambda i: (i, 0)),
        ],
        out_shape=[jax.ShapeDtypeStruct((NN, 1), _F32),
                   jax.ShapeDtypeStruct((NN, 1), _F32),
                   jax.ShapeDtypeStruct((NN, HH), _F32)],
    )(od, idg, xw)


def _tc_mid(p, nd, b, ns, W):
    def body(p_ref, nd_ref, b_ref, ns_ref, w_ref, o_ref):
        pr = p_ref[...]
        h = jnp.maximum((pr[0] + pr[1]) * nd_ref[...] + b_ref[...], 0.0)
        o_ref[...] = _dot3(h * ns_ref[...], w_ref[...])

    return pl.pallas_call(
        body,
        grid=(NBLK,),
        in_specs=[
            pl.BlockSpec((NCORES, RB, HH), lambda i: (0, i, 0)),
            pl.BlockSpec((RB, 1), lambda i: (i, 0)),
            pl.BlockSpec((1, HH), lambda i: (0, 0)),
            pl.BlockSpec((RB, 1), lambda i: (i, 0)),
            pl.BlockSpec((HH, HH), lambda i: (0, 0)),
        ],
        out_specs=pl.BlockSpec((RB, HH), lambda i: (i, 0)),
        out_shape=jax.ShapeDtypeStruct((NN, HH), _F32),
    )(p, nd, b, ns, W)


def _tc_final(p, nd, b, w1, b1, g, be, w2, b2):
    def body(p_ref, nd_ref, b_ref, w1_ref, b1_ref, g_ref, be_ref, w2_ref,
             b2_ref, o_ref, sacc, macc):
        i = pl.program_id(0)
        pr = p_ref[...]
        h = jnp.maximum((pr[0] + pr[1]) * nd_ref[...] + b_ref[...], 0.0)
        bs = jnp.sum(h, axis=0, keepdims=True)
        bm = jnp.max(h, axis=0, keepdims=True)

        @pl.when(i == 0)
        def _():
            sacc[...] = bs
            macc[...] = bm

        @pl.when(i > 0)
        def _():
            sacc[...] += bs
            macc[...] = jnp.maximum(macc[...], bm)

        @pl.when(i == NBLK - 1)
        def _():
            rs = sacc[...]
            rm = rs * (1.0 / NN)
            rx = macc[...]
            w1r = w1_ref[...]
            z = (
                jnp.dot(rs, w1r[0:HH], preferred_element_type=_F32, precision=jax.lax.Precision.HIGHEST)
                + jnp.dot(rm, w1r[HH:2 * HH], preferred_element_type=_F32, precision=jax.lax.Precision.HIGHEST)
                + jnp.dot(rx, w1r[2 * HH:3 * HH], preferred_element_type=_F32, precision=jax.lax.Precision.HIGHEST)
                + b1_ref[...]
            )
            z = z * (g_ref[...] * float(1.0 / np.sqrt(1.0 + EPSV))) + be_ref[...]
            z = jnp.maximum(z, 0.0)
            o_ref[...] = (
                jnp.dot(z, w2_ref[...], preferred_element_type=_F32, precision=jax.lax.Precision.HIGHEST)
                + b2_ref[...]
            )

    return pl.pallas_call(
        body,
        grid=(NBLK,),
        in_specs=[
            pl.BlockSpec((NCORES, RB, HH), lambda i: (0, i, 0)),
            pl.BlockSpec((RB, 1), lambda i: (i, 0)),
            pl.BlockSpec((1, HH), lambda i: (0, 0)),
            pl.BlockSpec((3 * HH, HH), lambda i: (0, 0)),
            pl.BlockSpec((1, HH), lambda i: (0, 0)),
            pl.BlockSpec((1, HH), lambda i: (0, 0)),
            pl.BlockSpec((1, HH), lambda i: (0, 0)),
            pl.BlockSpec((HH, 1), lambda i: (0, 0)),
            pl.BlockSpec((1, 1), lambda i: (0, 0)),
        ],
        out_specs=pl.BlockSpec((1, 1), lambda i: (0, 0)),
        out_shape=jax.ShapeDtypeStruct((1, 1), _F32),
        scratch_shapes=[
            pltpu.VMEM((1, HH), _F32),
            pltpu.VMEM((1, HH), _F32),
        ],
    )(p, nd, b, w1, b1, g, be, w2, b2)


def kernel(x, edge_index, W0, b0, W1, b1, W2, b2, W3, b3, W4, b4,
           mlpW1, mlpb1, gamma, beta, mlpW2, mlpb2):
    zvec = jnp.zeros((NN,), _F32)
    zrows = jnp.zeros((NACC, HH), _F32)
    pad = jnp.zeros((NCHP * CHUNK - EE,), jnp.int32)
    src2d = jnp.concatenate([edge_index[0], pad]).reshape(NCHP, CHUNK)
    dst2d = jnp.concatenate([edge_index[1], pad]).reshape(NCHP, CHUNK)

    xw = _tc_xw(x, W0)
    od, idg = _sc_degrees(src2d, dst2d, zvec)
    ns, nd, m = _tc_norms_scale(od.reshape(NN, 1), idg.reshape(NN, 1), xw)

    Ws = [W0, W1, W2, W3, W4]
    bs = [b0.reshape(1, HH), b1.reshape(1, HH), b2.reshape(1, HH),
          b3.reshape(1, HH), b4.reshape(1, HH)]

    p = None
    for l in range(5):
        p = _sc_aggregate(m, src2d, dst2d, zrows)
        if l < 4:
            m = _tc_mid(p, nd, bs[l], ns, Ws[l + 1])

    return _tc_final(
        p, nd, bs[4], mlpW1, mlpb1.reshape(1, HH), gamma.reshape(1, HH),
        beta.reshape(1, HH), mlpW2, mlpb2.reshape(1, 1),
    )
